# baseline (speedup 1.0000x reference)
import functools

import jax
import jax.numpy as jnp
from jax import lax
from jax.experimental import pallas as pl
from jax.experimental.pallas import tpu as pltpu
from jax.experimental.pallas import tpu_sc as plsc

B, N, K, D = 8, 256, 3, 32
NUM_EDGE, NUM_SPATIAL = 1536, 512
NP1 = N + 1
NC, NS, L = 2, 16, 16
NW = NC * NS
RPW = (B * N) // NW
WPB = NW // B
HALF = N // 2
UNITS = RPW * 2
THIRD = 1.0 / 3.0


def _body(sp_hbm, ef_hbm, spw_hbm, ew_hbm, vn_hbm, out_hbm,
          sp_tab, ed_tab, ob0, ob1, vrow, vnb,
          spv0, spv1, efv0, efv1,
          so0, so1, ss0, ss1, se0, se1):
    cid = lax.axis_index("c")
    sid = lax.axis_index("s")
    wid = sid * NC + cid
    b = wid // WPB
    i0 = (wid % WPB) * RPW + 1

    pltpu.sync_copy(spw_hbm, sp_tab)
    pltpu.sync_copy(ew_hbm, ed_tab)
    pltpu.sync_copy(vn_hbm, vnb)
    v0 = vnb[pl.ds(0, L)]
    v1 = vnb[pl.ds(L, L)]

    for ob in (ob0, ob1):
        ob[pl.ds(0, L)] = v0
        ob[pl.ds(L, L)] = v1

    @pl.when(wid < B)
    def _():
        def fill(j, c):
            vrow[pl.ds(j * D, L)] = v0
            vrow[pl.ds(j * D + L, L)] = v1
            return c
        lax.fori_loop(0, NP1, fill, 0)
        pltpu.sync_copy(vrow, out_hbm.at[wid, 0])

    obufs = (ob0, ob1)
    spss = (spv0, spv1)
    efss = (efv0, efv1)
    sems_o = (so0, so1)
    sems_s = (ss0, ss1)
    sems_e = (se0, se1)

    spvs = (spv0, spv1)
    efvs = (efv0, efv1)

    def start_idx(u, s):
        t = u // 2
        h = u % 2
        src_row = i0 - 1 + t
        pltpu.async_copy(sp_hbm.at[b, src_row, pl.ds(h * HALF, HALF)],
                         spvs[s], sems_s[s])
        pltpu.async_copy(ef_hbm.at[b, src_row, pl.ds(h * HALF * K, HALF * K)],
                         efvs[s], sems_e[s])

    def wait_idx(s):
        pltpu.make_async_copy(sp_hbm.at[b, 0, pl.ds(0, HALF)],
                              spvs[s], sems_s[s]).wait()
        pltpu.make_async_copy(ef_hbm.at[b, 0, pl.ds(0, HALF * K)],
                              efvs[s], sems_e[s]).wait()

    def wait_out(s):
        pltpu.make_async_copy(obufs[s], out_hbm.at[b, 1], sems_o[s]).wait()

    start_idx(0, 0)

    def compute(ob, sps, efs, half):
        base_col = half * HALF + 1

        @plsc.parallel_loop(0, HALF // L, unroll=4)
        def _grp(g):
            spv = sps[pl.ds(g * L, L)]
            evs = (efs[pl.ds(g * L * K, L)],
                   efs[pl.ds(g * L * K + L, L)],
                   efs[pl.ds(g * L * K + 2 * L, L)])
            ob_base = (base_col + g * L) * D
            for pp in range(L):
                f = K * pp
                rb = spv[pp] * D
                b0 = evs[f // L][f % L] * D
                b1 = evs[(f + 1) // L][(f + 1) % L] * D
                b2 = evs[(f + 2) // L][(f + 2) % L] * D
                po = ob_base + pp * D
                for h in range(2):
                    o = h * L
                    val = (sp_tab[pl.ds(rb + o, L)]
                           + (ed_tab[pl.ds(b0 + o, L)]
                              + ed_tab[pl.ds(b1 + o, L)]
                              + ed_tab[pl.ds(b2 + o, L)]) * THIRD)
                    ob[pl.ds(po + o, L)] = val

    def iter_body(rr, c):
        for sub in range(4):
            u = rr * 4 + sub
            t = rr * 2 + sub // 2
            s_idx = sub % 2
            s_ob = sub // 2

            @pl.when(u + 1 < UNITS)
            def _():
                start_idx(u + 1, 1 - s_idx)

            wait_idx(s_idx)

            if sub % 2 == 0:
                @pl.when(t >= 2)
                def _():
                    wait_out(s_ob)

            compute(obufs[s_ob], spss[s_idx], efss[s_idx], s_idx)

            if sub % 2 == 1:
                pltpu.async_copy(obufs[s_ob], out_hbm.at[b, i0 + t],
                                 sems_o[s_ob])
        return c

    lax.fori_loop(0, RPW // 2, iter_body, 0)
    wait_out(0)
    wait_out(1)


_edge_kernel = functools.partial(
    pl.kernel,
    out_type=jax.ShapeDtypeStruct((B, NP1, NP1 * D), jnp.float32),
    mesh=plsc.VectorSubcoreMesh(core_axis_name="c", subcore_axis_name="s",
                                num_cores=NC, num_subcores=NS),
    compiler_params=pltpu.CompilerParams(needs_layout_passes=False),
    scratch_types=[
        pltpu.VMEM((NUM_SPATIAL * D,), jnp.float32),
        pltpu.VMEM((NUM_EDGE * D,), jnp.float32),
        pltpu.VMEM((NP1 * D,), jnp.float32),
        pltpu.VMEM((NP1 * D,), jnp.float32),
        pltpu.VMEM((NP1 * D,), jnp.float32),
        pltpu.VMEM((D,), jnp.float32),
        pltpu.VMEM((HALF,), jnp.int32),
        pltpu.VMEM((HALF,), jnp.int32),
        pltpu.VMEM((HALF * K,), jnp.int32),
        pltpu.VMEM((HALF * K,), jnp.int32),
        pltpu.SemaphoreType.DMA,
        pltpu.SemaphoreType.DMA,
        pltpu.SemaphoreType.DMA,
        pltpu.SemaphoreType.DMA,
        pltpu.SemaphoreType.DMA,
        pltpu.SemaphoreType.DMA,
    ],
)(_body)


def kernel(graph_attn_bias, shortest_path, edge_feat, sp_weight, edge_weight, vnode_weight):
    del graph_attn_bias
    out = _edge_kernel(
        shortest_path.astype(jnp.int32),
        edge_feat.astype(jnp.int32).reshape(B, N, N * K),
        sp_weight.astype(jnp.float32).reshape(NUM_SPATIAL * D),
        edge_weight.astype(jnp.float32).reshape(NUM_EDGE * D),
        vnode_weight.astype(jnp.float32).reshape(D),
    )
    return out.reshape(B, NP1, NP1, D)


# speedup vs baseline: 1.2214x; 1.2214x over previous
import functools

import jax
import jax.numpy as jnp
from jax import lax
from jax.experimental import pallas as pl
from jax.experimental.pallas import tpu as pltpu
from jax.experimental.pallas import tpu_sc as plsc

B, N, K, D = 8, 256, 3, 32
NUM_EDGE, NUM_SPATIAL = 1536, 512
NP1 = N + 1
NC, NS, L = 2, 16, 16
NW = NC * NS
RPW = (B * N) // NW
WPB = NW // B
CHUNK = N
UNITS = RPW
THIRD = 1.0 / 3.0


def _body(sp_hbm, ef_hbm, spw_hbm, ew_hbm, vn_hbm, out_hbm,
          sp_tab, ed_tab, ob0, ob1, vrow, vnb,
          spv0, spv1, efv0, efv1,
          so0, so1, ss0, ss1, se0, se1):
    cid = lax.axis_index("c")
    sid = lax.axis_index("s")
    wid = sid * NC + cid
    b = wid // WPB
    i0 = (wid % WPB) * RPW + 1

    pltpu.sync_copy(spw_hbm, sp_tab)
    pltpu.sync_copy(ew_hbm, ed_tab)
    pltpu.sync_copy(vn_hbm, vnb)
    v0 = vnb[pl.ds(0, L)]
    v1 = vnb[pl.ds(L, L)]

    for ob in (ob0, ob1):
        ob[pl.ds(0, L)] = v0
        ob[pl.ds(L, L)] = v1

    @pl.when(wid < B)
    def _():
        def fill(j, c):
            vrow[pl.ds(j * D, L)] = v0
            vrow[pl.ds(j * D + L, L)] = v1
            return c
        lax.fori_loop(0, NP1, fill, 0)
        pltpu.sync_copy(vrow, out_hbm.at[wid, 0])

    obufs = (ob0, ob1)
    spss = (spv0, spv1)
    efss = (efv0, efv1)
    sems_o = (so0, so1)
    sems_s = (ss0, ss1)
    sems_e = (se0, se1)

    spvs = (spv0, spv1)
    efvs = (efv0, efv1)

    def start_idx(u, s):
        src_row = i0 - 1 + u
        pltpu.async_copy(sp_hbm.at[b, src_row], spvs[s], sems_s[s])
        pltpu.async_copy(ef_hbm.at[b, src_row], efvs[s], sems_e[s])

    def wait_idx(s):
        pltpu.make_async_copy(sp_hbm.at[b, 0], spvs[s], sems_s[s]).wait()
        pltpu.make_async_copy(ef_hbm.at[b, 0], efvs[s], sems_e[s]).wait()

    def wait_out(s):
        pltpu.make_async_copy(obufs[s], out_hbm.at[b, 1], sems_o[s]).wait()

    start_idx(0, 0)

    def compute(ob, sps, efs):
        base_col = 1

        @plsc.parallel_loop(0, CHUNK // L, unroll=2)
        def _grp(g):
            spv = sps[pl.ds(g * L, L)]
            evs = (efs[pl.ds(g * L * K, L)],
                   efs[pl.ds(g * L * K + L, L)],
                   efs[pl.ds(g * L * K + 2 * L, L)])
            ob_base = (base_col + g * L) * D
            for pp in range(L):
                f = K * pp
                rb = spv[pp] * D
                b0 = evs[f // L][f % L] * D
                b1 = evs[(f + 1) // L][(f + 1) % L] * D
                b2 = evs[(f + 2) // L][(f + 2) % L] * D
                po = ob_base + pp * D
                for h in range(2):
                    o = h * L
                    val = (sp_tab[pl.ds(rb + o, L)]
                           + (ed_tab[pl.ds(b0 + o, L)]
                              + ed_tab[pl.ds(b1 + o, L)]
                              + ed_tab[pl.ds(b2 + o, L)]) * THIRD)
                    ob[pl.ds(po + o, L)] = val

    def iter_body(rr, c):
        for sub in range(2):
            t = rr * 2 + sub
            s = sub

            @pl.when(t + 1 < UNITS)
            def _():
                start_idx(t + 1, 1 - s)

            wait_idx(s)

            @pl.when(t >= 2)
            def _():
                wait_out(s)

            compute(obufs[s], spss[s], efss[s])
            pltpu.async_copy(obufs[s], out_hbm.at[b, i0 + t], sems_o[s])
        return c

    lax.fori_loop(0, RPW // 2, iter_body, 0)
    wait_out(0)
    wait_out(1)


_edge_kernel = functools.partial(
    pl.kernel,
    out_type=jax.ShapeDtypeStruct((B, NP1, NP1 * D), jnp.float32),
    mesh=plsc.VectorSubcoreMesh(core_axis_name="c", subcore_axis_name="s",
                                num_cores=NC, num_subcores=NS),
    compiler_params=pltpu.CompilerParams(needs_layout_passes=False),
    scratch_types=[
        pltpu.VMEM((NUM_SPATIAL * D,), jnp.float32),
        pltpu.VMEM((NUM_EDGE * D,), jnp.float32),
        pltpu.VMEM((NP1 * D,), jnp.float32),
        pltpu.VMEM((NP1 * D,), jnp.float32),
        pltpu.VMEM((NP1 * D,), jnp.float32),
        pltpu.VMEM((D,), jnp.float32),
        pltpu.VMEM((CHUNK,), jnp.int32),
        pltpu.VMEM((CHUNK,), jnp.int32),
        pltpu.VMEM((CHUNK * K,), jnp.int32),
        pltpu.VMEM((CHUNK * K,), jnp.int32),
        pltpu.SemaphoreType.DMA,
        pltpu.SemaphoreType.DMA,
        pltpu.SemaphoreType.DMA,
        pltpu.SemaphoreType.DMA,
        pltpu.SemaphoreType.DMA,
        pltpu.SemaphoreType.DMA,
    ],
)(_body)


def kernel(graph_attn_bias, shortest_path, edge_feat, sp_weight, edge_weight, vnode_weight):
    del graph_attn_bias
    out = _edge_kernel(
        shortest_path.astype(jnp.int32),
        edge_feat.astype(jnp.int32).reshape(B, N, N * K),
        sp_weight.astype(jnp.float32).reshape(NUM_SPATIAL * D),
        edge_weight.astype(jnp.float32).reshape(NUM_EDGE * D),
        vnode_weight.astype(jnp.float32).reshape(D),
    )
    return out.reshape(B, NP1, NP1, D)


# speedup vs baseline: 1.2702x; 1.0400x over previous
import functools

import jax
import jax.numpy as jnp
from jax import lax
from jax.experimental import pallas as pl
from jax.experimental.pallas import tpu as pltpu
from jax.experimental.pallas import tpu_sc as plsc

B, N, K, D = 8, 256, 3, 32
NUM_EDGE, NUM_SPATIAL = 1536, 512
NP1 = N + 1
NC, NS, L = 2, 16, 16
NW = NC * NS
RPW = (B * N) // NW
WPB = NW // B
CHUNK = N
UNITS = RPW
THIRD = 1.0 / 3.0


def _body(sp_hbm, ef_hbm, spw_hbm, ew_hbm, vn_hbm, out_hbm,
          sp_tab, ed_tab, ob0, ob1, vrow, vnb,
          spv0, spv1, efv0, efv1,
          so0, so1, ss0, ss1, se0, se1):
    cid = lax.axis_index("c")
    sid = lax.axis_index("s")
    wid = sid * NC + cid
    b = wid // WPB
    i0 = (wid % WPB) * RPW + 1

    pltpu.sync_copy(spw_hbm, sp_tab)
    pltpu.sync_copy(ew_hbm, ed_tab)
    pltpu.sync_copy(vn_hbm, vnb)
    v0 = vnb[pl.ds(0, L)]
    v1 = vnb[pl.ds(L, L)]

    for ob in (ob0, ob1):
        ob[pl.ds(0, L)] = v0
        ob[pl.ds(L, L)] = v1

    @pl.when(wid < B)
    def _():
        def fill(j, c):
            vrow[pl.ds(j * D, L)] = v0
            vrow[pl.ds(j * D + L, L)] = v1
            return c
        lax.fori_loop(0, NP1, fill, 0)
        pltpu.sync_copy(vrow, out_hbm.at[wid, 0])

    obufs = (ob0, ob1)
    spss = (spv0, spv1)
    efss = (efv0, efv1)
    sems_o = (so0, so1)
    sems_s = (ss0, ss1)
    sems_e = (se0, se1)

    spvs = (spv0, spv1)
    efvs = (efv0, efv1)

    def start_idx(u, s):
        src_row = i0 - 1 + u
        pltpu.async_copy(sp_hbm.at[b, src_row], spvs[s], sems_s[s])
        pltpu.async_copy(ef_hbm.at[b, src_row], efvs[s], sems_e[s])

    def wait_idx(s):
        pltpu.make_async_copy(sp_hbm.at[b, 0], spvs[s], sems_s[s]).wait()
        pltpu.make_async_copy(ef_hbm.at[b, 0], efvs[s], sems_e[s]).wait()

    def wait_out(s):
        pltpu.make_async_copy(obufs[s], out_hbm.at[b, 1], sems_o[s]).wait()

    start_idx(0, 0)

    def compute(ob, sps, efs):
        base_col = 1

        @plsc.parallel_loop(0, CHUNK // L, unroll=2)
        def _grp(g):
            spv = sps[pl.ds(g * L, L)] * D
            evs = (efs[pl.ds(g * L * K, L)] * D,
                   efs[pl.ds(g * L * K + L, L)] * D,
                   efs[pl.ds(g * L * K + 2 * L, L)] * D)
            ob_base = (base_col + g * L) * D
            for pp in range(L):
                f = K * pp
                rb = spv[pp]
                b0 = evs[f // L][f % L]
                b1 = evs[(f + 1) // L][(f + 1) % L]
                b2 = evs[(f + 2) // L][(f + 2) % L]
                po = ob_base + pp * D
                for h in range(2):
                    o = h * L
                    val = (sp_tab[pl.ds(rb + o, L)]
                           + (ed_tab[pl.ds(b0 + o, L)]
                              + ed_tab[pl.ds(b1 + o, L)]
                              + ed_tab[pl.ds(b2 + o, L)]) * THIRD)
                    ob[pl.ds(po + o, L)] = val

    def iter_body(rr, c):
        for sub in range(2):
            t = rr * 2 + sub
            s = sub

            @pl.when(t + 1 < UNITS)
            def _():
                start_idx(t + 1, 1 - s)

            wait_idx(s)

            @pl.when(t >= 2)
            def _():
                wait_out(s)

            compute(obufs[s], spss[s], efss[s])
            pltpu.async_copy(obufs[s], out_hbm.at[b, i0 + t], sems_o[s])
        return c

    lax.fori_loop(0, RPW // 2, iter_body, 0)
    wait_out(0)
    wait_out(1)


_edge_kernel = functools.partial(
    pl.kernel,
    out_type=jax.ShapeDtypeStruct((B, NP1, NP1 * D), jnp.float32),
    mesh=plsc.VectorSubcoreMesh(core_axis_name="c", subcore_axis_name="s",
                                num_cores=NC, num_subcores=NS),
    compiler_params=pltpu.CompilerParams(needs_layout_passes=False),
    scratch_types=[
        pltpu.VMEM((NUM_SPATIAL * D,), jnp.float32),
        pltpu.VMEM((NUM_EDGE * D,), jnp.float32),
        pltpu.VMEM((NP1 * D,), jnp.float32),
        pltpu.VMEM((NP1 * D,), jnp.float32),
        pltpu.VMEM((NP1 * D,), jnp.float32),
        pltpu.VMEM((D,), jnp.float32),
        pltpu.VMEM((CHUNK,), jnp.int32),
        pltpu.VMEM((CHUNK,), jnp.int32),
        pltpu.VMEM((CHUNK * K,), jnp.int32),
        pltpu.VMEM((CHUNK * K,), jnp.int32),
        pltpu.SemaphoreType.DMA,
        pltpu.SemaphoreType.DMA,
        pltpu.SemaphoreType.DMA,
        pltpu.SemaphoreType.DMA,
        pltpu.SemaphoreType.DMA,
        pltpu.SemaphoreType.DMA,
    ],
)(_body)


def kernel(graph_attn_bias, shortest_path, edge_feat, sp_weight, edge_weight, vnode_weight):
    del graph_attn_bias
    out = _edge_kernel(
        shortest_path.astype(jnp.int32),
        edge_feat.astype(jnp.int32).reshape(B, N, N * K),
        sp_weight.astype(jnp.float32).reshape(NUM_SPATIAL * D),
        edge_weight.astype(jnp.float32).reshape(NUM_EDGE * D),
        vnode_weight.astype(jnp.float32).reshape(D),
    )
    return out.reshape(B, NP1, NP1, D)


# speedup vs baseline: 1.3662x; 1.0756x over previous
import functools

import jax
import jax.numpy as jnp
from jax import lax
from jax.experimental import pallas as pl
from jax.experimental.pallas import tpu as pltpu
from jax.experimental.pallas import tpu_sc as plsc

B, N, K, D = 8, 256, 3, 32
NUM_EDGE, NUM_SPATIAL = 1536, 512
NP1 = N + 1
NC, NS, L = 2, 16, 16
NW = NC * NS
RPW = (B * N) // NW
WPB = NW // B
CHUNK = N
UNITS = RPW
THIRD = 1.0 / 3.0


def _body(sp_hbm, ef_hbm, spw_hbm, ew_hbm, vn_hbm, out_hbm,
          sp_tab, ed_tab, ob0, ob1, vrow, vnb,
          spv0, spv1, efv0, efv1,
          so0, so1, ss0, ss1, se0, se1):
    cid = lax.axis_index("c")
    sid = lax.axis_index("s")
    wid = sid * NC + cid
    b = wid // WPB
    i0 = (wid % WPB) * RPW + 1

    pltpu.sync_copy(spw_hbm, sp_tab)
    pltpu.sync_copy(ew_hbm, ed_tab)
    pltpu.sync_copy(vn_hbm, vnb)
    v0 = vnb[pl.ds(0, L)]
    v1 = vnb[pl.ds(L, L)]

    for ob in (ob0, ob1):
        ob[pl.ds(0, L)] = v0
        ob[pl.ds(L, L)] = v1

    @pl.when(wid < B)
    def _():
        def fill(j, c):
            vrow[pl.ds(j * D, L)] = v0
            vrow[pl.ds(j * D + L, L)] = v1
            return c
        lax.fori_loop(0, NP1, fill, 0)
        pltpu.sync_copy(vrow, out_hbm.at[wid, 0])

    obufs = (ob0, ob1)
    spss = (spv0, spv1)
    efss = (efv0, efv1)
    sems_o = (so0, so1)
    sems_s = (ss0, ss1)
    sems_e = (se0, se1)

    spvs = (spv0, spv1)
    efvs = (efv0, efv1)

    def start_idx(u, s):
        src_row = i0 - 1 + u
        pltpu.async_copy(sp_hbm.at[b, src_row], spvs[s], sems_s[s])
        pltpu.async_copy(ef_hbm.at[b, src_row], efvs[s], sems_e[s])

    def wait_idx(s):
        pltpu.make_async_copy(sp_hbm.at[b, 0], spvs[s], sems_s[s]).wait()
        pltpu.make_async_copy(ef_hbm.at[b, 0], efvs[s], sems_e[s]).wait()

    def wait_out(s):
        pltpu.make_async_copy(obufs[s], out_hbm.at[b, 1], sems_o[s]).wait()

    start_idx(0, 0)

    def compute(ob, sps, efs):
        base_col = 1

        @plsc.parallel_loop(0, CHUNK // L, unroll=1)
        def _grp(g):
            spv = sps[pl.ds(g * L, L)] * D
            evs = (efs[pl.ds(g * L * K, L)] * D,
                   efs[pl.ds(g * L * K + L, L)] * D,
                   efs[pl.ds(g * L * K + 2 * L, L)] * D)
            ob_base = (base_col + g * L) * D
            for pp in range(L):
                f = K * pp
                rb = spv[pp]
                b0 = evs[f // L][f % L]
                b1 = evs[(f + 1) // L][(f + 1) % L]
                b2 = evs[(f + 2) // L][(f + 2) % L]
                po = ob_base + pp * D
                for h in range(2):
                    o = h * L
                    val = (sp_tab[pl.ds(rb + o, L)]
                           + (ed_tab[pl.ds(b0 + o, L)]
                              + ed_tab[pl.ds(b1 + o, L)]
                              + ed_tab[pl.ds(b2 + o, L)]) * THIRD)
                    ob[pl.ds(po + o, L)] = val

    def iter_body(rr, c):
        for sub in range(2):
            t = rr * 2 + sub
            s = sub

            @pl.when(t + 1 < UNITS)
            def _():
                start_idx(t + 1, 1 - s)

            wait_idx(s)

            @pl.when(t >= 2)
            def _():
                wait_out(s)

            compute(obufs[s], spss[s], efss[s])
            pltpu.async_copy(obufs[s], out_hbm.at[b, i0 + t], sems_o[s])
        return c

    lax.fori_loop(0, RPW // 2, iter_body, 0)
    wait_out(0)
    wait_out(1)


_edge_kernel = functools.partial(
    pl.kernel,
    out_type=jax.ShapeDtypeStruct((B, NP1, NP1 * D), jnp.float32),
    mesh=plsc.VectorSubcoreMesh(core_axis_name="c", subcore_axis_name="s",
                                num_cores=NC, num_subcores=NS),
    compiler_params=pltpu.CompilerParams(needs_layout_passes=False),
    scratch_types=[
        pltpu.VMEM((NUM_SPATIAL * D,), jnp.float32),
        pltpu.VMEM((NUM_EDGE * D,), jnp.float32),
        pltpu.VMEM((NP1 * D,), jnp.float32),
        pltpu.VMEM((NP1 * D,), jnp.float32),
        pltpu.VMEM((NP1 * D,), jnp.float32),
        pltpu.VMEM((D,), jnp.float32),
        pltpu.VMEM((CHUNK,), jnp.int32),
        pltpu.VMEM((CHUNK,), jnp.int32),
        pltpu.VMEM((CHUNK * K,), jnp.int32),
        pltpu.VMEM((CHUNK * K,), jnp.int32),
        pltpu.SemaphoreType.DMA,
        pltpu.SemaphoreType.DMA,
        pltpu.SemaphoreType.DMA,
        pltpu.SemaphoreType.DMA,
        pltpu.SemaphoreType.DMA,
        pltpu.SemaphoreType.DMA,
    ],
)(_body)


def kernel(graph_attn_bias, shortest_path, edge_feat, sp_weight, edge_weight, vnode_weight):
    del graph_attn_bias
    out = _edge_kernel(
        shortest_path.astype(jnp.int32),
        edge_feat.astype(jnp.int32).reshape(B, N, N * K),
        sp_weight.astype(jnp.float32).reshape(NUM_SPATIAL * D),
        edge_weight.astype(jnp.float32).reshape(NUM_EDGE * D),
        vnode_weight.astype(jnp.float32).reshape(D),
    )
    return out.reshape(B, NP1, NP1, D)
